# baseline (speedup 1.0000x reference)
import functools

import jax
import jax.numpy as jnp
from jax import lax
from jax.experimental import pallas as pl
from jax.experimental.pallas import tpu as pltpu
from jax.experimental.pallas import tpu_sc as plsc

VOCAB_SIZE = 100000
EMBED_DIM = 128
MAXLEN = 200
BATCH = 1024

NUM_CORES = 2
NUM_SUBCORES = 16
NUM_WORKERS = NUM_CORES * NUM_SUBCORES

SUBGATHER = 100
SUBS = MAXLEN // SUBGATHER
N = BATCH // NUM_WORKERS
NBUF = 4
NGROUPS = N // NBUF
LANES = 16
VECS_PER_ROW = EMBED_DIM // LANES

STORE_LO = 96
STORE_HI = MAXLEN - STORE_LO


def _emb_kernel(idx_hbm, token_hbm, pos_hbm, out_hbm, pos_v,
                ibuf0, ibuf1, rows0, rows1, rows2, rows3,
                isem0, isem1,
                gsem00, gsem01, gsem10, gsem11,
                gsem20, gsem21, gsem30, gsem31,
                ssem0, ssem1, ssem2, ssem3, psem):
    ibuf = (ibuf0, ibuf1)
    isem = (isem0, isem1)
    rows_b = (rows0, rows1, rows2, rows3)
    gsem_b = ((gsem00, gsem01), (gsem10, gsem11),
              (gsem20, gsem21), (gsem30, gsem31))
    ssem_b = (ssem0, ssem1, ssem2, ssem3)

    wid = lax.axis_index("s") * NUM_CORES + lax.axis_index("c")
    base = wid * N

    pltpu.async_copy(pos_hbm, pos_v, psem)

    def load_group(G, gpar):
        return pltpu.async_copy(
            idx_hbm.at[pl.ds(base + NBUF * G, NBUF)], ibuf[gpar], isem[gpar]
        )

    def wait_group(gpar):
        pltpu.make_async_copy(
            idx_hbm.at[pl.ds(base, NBUF)], ibuf[gpar], isem[gpar]
        ).wait()

    def gather_half(gpar, p, h):
        return (
            token_hbm.at[ibuf[gpar].at[p, h]],
            rows_b[p].at[pl.ds(h * SUBGATHER, SUBGATHER)],
            gsem_b[p][h],
        )

    def start_gathers(gpar, p):
        for h in range(SUBS):
            pltpu.async_copy(*gather_half(gpar, p, h))

    def wait_gather_half(gpar, p, h):
        pltpu.make_async_copy(*gather_half(gpar, p, h)).wait()

    def wait_store(p):
        pltpu.make_async_copy(
            rows_b[p], out_hbm.at[pl.ds(0, MAXLEN)], ssem_b[p]
        ).wait()

    def add_pos_rows(p, lo, n):
        def row_body(r, c2):
            for v in range(VECS_PER_ROW):
                sl = pl.ds(v * LANES, LANES)
                plsc.addupdate(rows_b[p].at[r, sl], pos_v[r, sl])
            return c2

        lax.fori_loop(lo, lo + n, row_body, 0, unroll=False)

    def store_rows(p, j, lo, n):
        pltpu.async_copy(
            rows_b[p].at[pl.ds(lo, n)],
            out_hbm.at[pl.ds((base + j) * MAXLEN + lo, n)],
            ssem_b[p],
        )

    load_group(0, 0)
    load_group(1, 1)
    wait_group(0)
    start_gathers(0, 0)
    start_gathers(0, 1)
    pltpu.make_async_copy(pos_hbm, pos_v, psem).wait()

    def pair_body(k, carry):
        for g in range(2):
            G = 2 * k + g
            gq = g ^ 1
            for p in range(NBUF):
                j = NBUF * G + p
                g2 = (p + 2) % NBUF

                if p == 0:
                    @pl.when((j + NBUF < N) & (j >= NBUF))
                    def _():
                        load_group(G + 1, gq)

                @pl.when(j >= 2)
                def _():
                    wait_store(g2)

                if p < 2:
                    @pl.when(j + 2 < N)
                    def _():
                        start_gathers(g, g2)
                else:
                    if p == 2:
                        @pl.when(j + 2 < N)
                        def _():
                            wait_group(gq)
                            start_gathers(gq, g2)
                    else:
                        @pl.when(j + 2 < N)
                        def _():
                            start_gathers(gq, g2)

                for h in range(SUBS):
                    wait_gather_half(g, p, h)
                    if h == 0:
                        add_pos_rows(p, 0, STORE_LO)
                        store_rows(p, j, 0, STORE_LO)
                    else:
                        add_pos_rows(p, STORE_LO, STORE_HI)
                        store_rows(p, j, STORE_LO, STORE_HI)
        return carry

    lax.fori_loop(0, NGROUPS // 2, pair_body, 0, unroll=False)
    wait_store((N - 2) % NBUF)
    wait_store((N - 1) % NBUF)


@functools.partial(jax.jit, static_argnames=())
def kernel(inputs, token_table, pos_table):
    idx = inputs.reshape(BATCH, SUBS, SUBGATHER).astype(jnp.int32)
    mesh = plsc.VectorSubcoreMesh(core_axis_name="c", subcore_axis_name="s")
    scratch = [pltpu.VMEM((MAXLEN, EMBED_DIM), jnp.float32)]
    scratch += [pltpu.VMEM((NBUF, SUBS, SUBGATHER), jnp.int32)] * 2
    scratch += [pltpu.VMEM((MAXLEN, EMBED_DIM), jnp.float32)] * NBUF
    scratch += [pltpu.SemaphoreType.DMA] * 2
    scratch += [pltpu.SemaphoreType.DMA] * (2 * NBUF)
    scratch += [pltpu.SemaphoreType.DMA] * NBUF
    scratch += [pltpu.SemaphoreType.DMA]
    out = pl.kernel(
        _emb_kernel,
        mesh=mesh,
        out_type=jax.ShapeDtypeStruct((BATCH * MAXLEN, EMBED_DIM), jnp.float32),
        scratch_types=scratch,
    )(idx, token_table, pos_table)
    return out.reshape(BATCH, MAXLEN, EMBED_DIM)


# speedup vs baseline: 1.0075x; 1.0075x over previous
import functools

import jax
import jax.numpy as jnp
from jax import lax
from jax.experimental import pallas as pl
from jax.experimental.pallas import tpu as pltpu
from jax.experimental.pallas import tpu_sc as plsc

VOCAB_SIZE = 100000
EMBED_DIM = 128
MAXLEN = 200
BATCH = 1024

NUM_CORES = 2
NUM_SUBCORES = 16
NUM_WORKERS = NUM_CORES * NUM_SUBCORES

SUBGATHER = 100
SUBS = MAXLEN // SUBGATHER
N = BATCH // NUM_WORKERS
NBUF = 4
LANES = 16
VECS_PER_ROW = EMBED_DIM // LANES

STORE_LO = 96
STORE_HI = MAXLEN - STORE_LO


def _emb_kernel(idx_hbm, token_hbm, pos_hbm, out_hbm, pos_v, *rest):
    idx_b = rest[0:NBUF]
    rows_b = rest[NBUF:2 * NBUF]
    isem_b = rest[2 * NBUF:3 * NBUF]
    gsem_b = tuple(
        tuple(rest[3 * NBUF + 2 * b:3 * NBUF + 2 * b + 2]) for b in range(NBUF)
    )
    ssem_b = rest[5 * NBUF:6 * NBUF]
    psem = rest[6 * NBUF]

    wid = lax.axis_index("s") * NUM_CORES + lax.axis_index("c")
    base = wid * N

    pltpu.async_copy(pos_hbm, pos_v, psem)

    def gather_half(p, h):
        return (
            token_hbm.at[idx_b[p].at[h]],
            rows_b[p].at[pl.ds(h * SUBGATHER, SUBGATHER)],
            gsem_b[p][h],
        )

    def start_gathers(p):
        for h in range(SUBS):
            pltpu.async_copy(*gather_half(p, h))

    def wait_gather_half(p, h):
        pltpu.make_async_copy(*gather_half(p, h)).wait()

    def wait_idx(p):
        pltpu.make_async_copy(idx_hbm.at[base], idx_b[p], isem_b[p]).wait()

    def wait_store(p):
        pltpu.make_async_copy(
            rows_b[p], out_hbm.at[pl.ds(0, MAXLEN)], ssem_b[p]
        ).wait()

    def add_pos_rows(p, lo, n):
        def row_body(r, c2):
            for v in range(VECS_PER_ROW):
                sl = pl.ds(v * LANES, LANES)
                plsc.addupdate(rows_b[p].at[r, sl], pos_v[r, sl])
            return c2

        lax.fori_loop(lo, lo + n, row_body, 0, unroll=False)

    def store_rows(p, j, lo, n):
        pltpu.async_copy(
            rows_b[p].at[pl.ds(lo, n)],
            out_hbm.at[pl.ds((base + j) * MAXLEN + lo, n)],
            ssem_b[p],
        )

    pltpu.async_copy(idx_hbm.at[base], idx_b[0], isem_b[0])
    pltpu.async_copy(idx_hbm.at[base + 1], idx_b[1], isem_b[1])
    wait_idx(0)
    start_gathers(0)
    wait_idx(1)
    start_gathers(1)
    pltpu.async_copy(idx_hbm.at[base + 2], idx_b[2], isem_b[2])
    pltpu.async_copy(idx_hbm.at[base + 3], idx_b[3], isem_b[3])
    pltpu.make_async_copy(pos_hbm, pos_v, psem).wait()

    def group_body(i, carry):
        for p in range(NBUF):
            j = NBUF * i + p
            pf = p
            g2 = (p + 2) % NBUF

            @pl.when(j >= 2)
            def _():
                wait_store(g2)

            @pl.when(j + 2 < N)
            def _():
                wait_idx(g2)
                start_gathers(g2)

            for h in range(SUBS):
                wait_gather_half(p, h)
                if h == 0:
                    add_pos_rows(p, 0, STORE_LO)
                    store_rows(p, j, 0, STORE_LO)
                else:
                    @pl.when(j + NBUF < N)
                    def _():
                        pltpu.async_copy(
                            idx_hbm.at[base + j + NBUF], idx_b[pf], isem_b[pf]
                        )

                    add_pos_rows(p, STORE_LO, STORE_HI)
                    store_rows(p, j, STORE_LO, STORE_HI)
        return carry

    lax.fori_loop(0, N // NBUF, group_body, 0, unroll=False)
    wait_store((N - 2) % NBUF)
    wait_store((N - 1) % NBUF)


@functools.partial(jax.jit, static_argnames=())
def kernel(inputs, token_table, pos_table):
    idx = inputs.reshape(BATCH, SUBS, SUBGATHER).astype(jnp.int32)
    mesh = plsc.VectorSubcoreMesh(core_axis_name="c", subcore_axis_name="s")
    scratch = [pltpu.VMEM((MAXLEN, EMBED_DIM), jnp.float32)]
    scratch += [pltpu.VMEM((SUBS, SUBGATHER), jnp.int32)] * NBUF
    scratch += [pltpu.VMEM((MAXLEN, EMBED_DIM), jnp.float32)] * NBUF
    scratch += [pltpu.SemaphoreType.DMA] * NBUF
    scratch += [pltpu.SemaphoreType.DMA] * (2 * NBUF)
    scratch += [pltpu.SemaphoreType.DMA] * NBUF
    scratch += [pltpu.SemaphoreType.DMA]
    out = pl.kernel(
        _emb_kernel,
        mesh=mesh,
        out_type=jax.ShapeDtypeStruct((BATCH * MAXLEN, EMBED_DIM), jnp.float32),
        scratch_types=scratch,
    )(idx, token_table, pos_table)
    return out.reshape(BATCH, MAXLEN, EMBED_DIM)
